# baseline (device time: 101243 ns/iter reference)
import jax
import jax.numpy as jnp
from jax import lax
from jax.experimental import pallas as pl
from jax.experimental.pallas import tpu as pltpu

N_DEV = 8
N_SUB = 8


def kernel(Q, K, V):
    b, s, h, d = Q.shape
    hd = h * d
    half = s // 2
    sub = half // N_SUB
    scale = d ** -0.5
    Q2 = Q.reshape(b, s, hd)
    K2 = K.reshape(b, s, hd)
    V2 = V.reshape(b, s, hd)
    s_glob = N_DEV * s

    def body(q_ref, k_ref, v_ref, out_ref, kg, vg, l_scr, qs,
             ksendA, krecvA, vsendA, vrecvA,
             ksendB, krecvB, vsendB, vrecvB):
        my = lax.axis_index("i")
        left = (my - 1) % N_DEV
        right = (my + 1) % N_DEV

        def rows_a(j, si):
            return pl.ds(j * s + si * sub, sub)

        def rows_b(j, si):
            return pl.ds(j * s + half + si * sub, sub)

        def desc_a(ref, j, si, send_sems, recv_sems, t):
            return pltpu.make_async_remote_copy(
                src_ref=ref.at[:, rows_a(j, si), :],
                dst_ref=ref.at[:, rows_a(j, si), :],
                send_sem=send_sems.at[t, si], recv_sem=recv_sems.at[t, si],
                device_id=(right,), device_id_type=pl.DeviceIdType.MESH)

        def desc_b(ref, j, si, send_sems, recv_sems, t):
            return pltpu.make_async_remote_copy(
                src_ref=ref.at[:, rows_b(j, si), :],
                dst_ref=ref.at[:, rows_b(j, si), :],
                send_sem=send_sems.at[t, si], recv_sem=recv_sems.at[t, si],
                device_id=(left,), device_id_type=pl.DeviceIdType.MESH)

        barrier = pltpu.get_barrier_semaphore()
        for nbr in (left, right):
            pl.semaphore_signal(barrier, inc=1, device_id=(nbr,),
                                device_id_type=pl.DeviceIdType.MESH)
        pl.semaphore_wait(barrier, 2)

        own_sends = []
        for si in range(N_SUB):
            for ref_, gref, sems in ((k_ref, kg, (ksendA, krecvA)),
                                     (v_ref, vg, (vsendA, vrecvA))):
                r = pltpu.make_async_remote_copy(
                    src_ref=ref_.at[:, pl.ds(si * sub, sub), :],
                    dst_ref=gref.at[:, rows_a(my, si), :],
                    send_sem=sems[0].at[0, si], recv_sem=sems[1].at[0, si],
                    device_id=(right,), device_id_type=pl.DeviceIdType.MESH)
                r.start()
                own_sends.append(r)
            for ref_, gref, sems in ((k_ref, kg, (ksendB, krecvB)),
                                     (v_ref, vg, (vsendB, vrecvB))):
                r = pltpu.make_async_remote_copy(
                    src_ref=ref_.at[:, pl.ds(half + si * sub, sub), :],
                    dst_ref=gref.at[:, rows_b(my, si), :],
                    send_sem=sems[0].at[0, si], recv_sem=sems[1].at[0, si],
                    device_id=(left,), device_id_type=pl.DeviceIdType.MESH)
                r.start()
                own_sends.append(r)

        out_ref[:, :, :] = jnp.zeros((b, s, hd), jnp.float32)
        l_scr[:, :, :] = jnp.zeros((b, s, hd), jnp.float32)
        qs[:, :, :] = q_ref[:, :, :] * scale

        def accum_block(bb, sl, kblk, vblk):
            q = qs[bb, :, sl]
            sb = lax.dot_general(q, kblk, (((1,), (1,)), ((), ())),
                                 preferred_element_type=jnp.float32)
            p = jnp.exp(sb)
            pv = lax.dot_general(p, vblk, (((1,), (0,)), ((), ())),
                                 preferred_element_type=jnp.float32)
            out_ref[bb, :, sl] = out_ref[bb, :, sl] + pv
            l_scr[bb, :, sl] = l_scr[bb, :, sl] + jnp.sum(p, axis=1,
                                                          keepdims=True)

        for bb in range(b):
            for hh in range(h):
                sl = slice(hh * d, (hh + 1) * d)
                accum_block(bb, sl, k_ref[bb, :, sl], v_ref[bb, :, sl])

        def process_hop(t, forward):
            jr = (my - t - 1) % N_DEV
            jl = (my + t + 1) % N_DEV
            for si in range(N_SUB):
                desc_a(kg, jr, si, ksendA, krecvA, t).wait_recv()
                desc_a(vg, jr, si, vsendA, vrecvA, t).wait_recv()
                desc_b(kg, jl, si, ksendB, krecvB, t).wait_recv()
                desc_b(vg, jl, si, vsendB, vrecvB, t).wait_recv()
                if forward:
                    desc_a(kg, jr, si, ksendA, krecvA, t + 1).start()
                    desc_a(vg, jr, si, vsendA, vrecvA, t + 1).start()
                    desc_b(kg, jl, si, ksendB, krecvB, t + 1).start()
                    desc_b(vg, jl, si, vsendB, vrecvB, t + 1).start()
            for bb in range(b):
                for hh in range(h):
                    sl = slice(hh * d, (hh + 1) * d)
                    accum_block(bb, sl,
                                kg[bb, pl.ds(jr * s, half), sl],
                                vg[bb, pl.ds(jr * s, half), sl])
                    accum_block(bb, sl,
                                kg[bb, pl.ds(jl * s + half, half), sl],
                                vg[bb, pl.ds(jl * s + half, half), sl])

        def hop_body(t, carry):
            process_hop(t, forward=True)
            return carry

        lax.fori_loop(0, N_DEV - 2, hop_body, 0)
        process_hop(N_DEV - 2, forward=False)

        for bb in range(b):
            out_ref[bb, :, :] = out_ref[bb, :, :] / l_scr[bb, :, :]

        for r in own_sends:
            r.wait_send()

        def drain_body(t, carry):
            jr = (my - t - 1) % N_DEV
            jl = (my + t + 1) % N_DEV
            for si in range(N_SUB):
                desc_a(kg, jr, si, ksendA, krecvA, t + 1).wait_send()
                desc_a(vg, jr, si, vsendA, vrecvA, t + 1).wait_send()
                desc_b(kg, jl, si, ksendB, krecvB, t + 1).wait_send()
                desc_b(vg, jl, si, vsendB, vrecvB, t + 1).wait_send()
            return carry

        lax.fori_loop(0, N_DEV - 2, drain_body, 0)

    sem = pltpu.SemaphoreType.DMA((N_DEV - 1, N_SUB))
    out = pl.pallas_call(
        body,
        out_shape=jax.ShapeDtypeStruct((b, s, hd), jnp.float32),
        in_specs=[pl.BlockSpec(memory_space=pltpu.VMEM)] * 3,
        out_specs=pl.BlockSpec(memory_space=pltpu.VMEM),
        scratch_shapes=[
            pltpu.VMEM((b, s_glob, hd), jnp.float32),
            pltpu.VMEM((b, s_glob, hd), jnp.float32),
            pltpu.VMEM((b, s, hd), jnp.float32),
            pltpu.VMEM((b, s, hd), jnp.float32),
            sem, sem, sem, sem,
            sem, sem, sem, sem,
        ],
        compiler_params=pltpu.CompilerParams(collective_id=0),
    )(Q2, K2, V2)
    return out.reshape(b, s, h, d)


# device time: 68037 ns/iter; 1.4881x vs baseline; 1.4881x over previous
import jax
import jax.numpy as jnp
from jax import lax
from jax.experimental import pallas as pl
from jax.experimental.pallas import tpu as pltpu

N_DEV = 8
N_SUB = 4


def kernel(Q, K, V):
    b, s, h, d = Q.shape
    hd = h * d
    half = s // 2
    sub = half // N_SUB
    scale = d ** -0.5
    Q2 = Q.reshape(b, s, hd)
    K2 = K.reshape(b, s, hd)
    V2 = V.reshape(b, s, hd)
    s_glob = N_DEV * s
    bf16 = jnp.bfloat16

    def body(q_ref, k_ref, v_ref, out_ref, kg, vg, l_scr, qs, kb, vb,
             ksendA, krecvA, vsendA, vrecvA,
             ksendB, krecvB, vsendB, vrecvB):
        my = lax.axis_index("i")
        left = (my - 1) % N_DEV
        right = (my + 1) % N_DEV

        def rows_a(j, si):
            return pl.ds(j * s + si * sub, sub)

        def rows_b(j, si):
            return pl.ds(j * s + half + si * sub, sub)

        def desc_a(ref, j, si, send_sems, recv_sems, t):
            return pltpu.make_async_remote_copy(
                src_ref=ref.at[:, rows_a(j, si), :],
                dst_ref=ref.at[:, rows_a(j, si), :],
                send_sem=send_sems.at[t, si], recv_sem=recv_sems.at[t, si],
                device_id=(right,), device_id_type=pl.DeviceIdType.MESH)

        def desc_b(ref, j, si, send_sems, recv_sems, t):
            return pltpu.make_async_remote_copy(
                src_ref=ref.at[:, rows_b(j, si), :],
                dst_ref=ref.at[:, rows_b(j, si), :],
                send_sem=send_sems.at[t, si], recv_sem=recv_sems.at[t, si],
                device_id=(left,), device_id_type=pl.DeviceIdType.MESH)

        barrier = pltpu.get_barrier_semaphore()
        for nbr in (left, right):
            pl.semaphore_signal(barrier, inc=1, device_id=(nbr,),
                                device_id_type=pl.DeviceIdType.MESH)
        pl.semaphore_wait(barrier, 2)

        kb[:, :, :] = k_ref[:, :, :].astype(bf16)
        vb[:, :, :] = v_ref[:, :, :].astype(bf16)

        own_sends = []
        for si in range(N_SUB):
            for ref_, gref, sems in ((kb, kg, (ksendA, krecvA)),
                                     (vb, vg, (vsendA, vrecvA))):
                r = pltpu.make_async_remote_copy(
                    src_ref=ref_.at[:, pl.ds(si * sub, sub), :],
                    dst_ref=gref.at[:, rows_a(my, si), :],
                    send_sem=sems[0].at[0, si], recv_sem=sems[1].at[0, si],
                    device_id=(right,), device_id_type=pl.DeviceIdType.MESH)
                r.start()
                own_sends.append(r)
            for ref_, gref, sems in ((kb, kg, (ksendB, krecvB)),
                                     (vb, vg, (vsendB, vrecvB))):
                r = pltpu.make_async_remote_copy(
                    src_ref=ref_.at[:, pl.ds(half + si * sub, sub), :],
                    dst_ref=gref.at[:, rows_b(my, si), :],
                    send_sem=sems[0].at[0, si], recv_sem=sems[1].at[0, si],
                    device_id=(left,), device_id_type=pl.DeviceIdType.MESH)
                r.start()
                own_sends.append(r)

        out_ref[:, :, :] = jnp.zeros((b, s, hd), jnp.float32)
        l_scr[:, :, :] = jnp.zeros((b, s, hd), jnp.float32)
        qs[:, :, :] = (q_ref[:, :, :] * scale).astype(bf16)

        def accum_block(bb, sl, kblk, vblk):
            q = qs[bb, :, sl]
            sb = lax.dot_general(q, kblk, (((1,), (1,)), ((), ())),
                                 preferred_element_type=jnp.float32)
            p = jnp.exp(sb)
            pv = lax.dot_general(p.astype(bf16), vblk,
                                 (((1,), (0,)), ((), ())),
                                 preferred_element_type=jnp.float32)
            out_ref[bb, :, sl] = out_ref[bb, :, sl] + pv
            l_scr[bb, :, sl] = l_scr[bb, :, sl] + jnp.sum(p, axis=1,
                                                          keepdims=True)

        for bb in range(b):
            for hh in range(h):
                sl = slice(hh * d, (hh + 1) * d)
                accum_block(bb, sl, kb[bb, :, sl], vb[bb, :, sl])

        def process_hop(t, forward):
            jr = (my - t - 1) % N_DEV
            jl = (my + t + 1) % N_DEV
            for si in range(N_SUB):
                desc_a(kg, jr, si, ksendA, krecvA, t).wait_recv()
                desc_a(vg, jr, si, vsendA, vrecvA, t).wait_recv()
                desc_b(kg, jl, si, ksendB, krecvB, t).wait_recv()
                desc_b(vg, jl, si, vsendB, vrecvB, t).wait_recv()
                if forward:
                    desc_a(kg, jr, si, ksendA, krecvA, t + 1).start()
                    desc_a(vg, jr, si, vsendA, vrecvA, t + 1).start()
                    desc_b(kg, jl, si, ksendB, krecvB, t + 1).start()
                    desc_b(vg, jl, si, vsendB, vrecvB, t + 1).start()
            for bb in range(b):
                for hh in range(h):
                    sl = slice(hh * d, (hh + 1) * d)
                    accum_block(bb, sl,
                                kg[bb, pl.ds(jr * s, half), sl],
                                vg[bb, pl.ds(jr * s, half), sl])
                    accum_block(bb, sl,
                                kg[bb, pl.ds(jl * s + half, half), sl],
                                vg[bb, pl.ds(jl * s + half, half), sl])

        def hop_body(t, carry):
            process_hop(t, forward=True)
            return carry

        lax.fori_loop(0, N_DEV - 2, hop_body, 0)
        process_hop(N_DEV - 2, forward=False)

        for bb in range(b):
            out_ref[bb, :, :] = out_ref[bb, :, :] / l_scr[bb, :, :]

        for r in own_sends:
            r.wait_send()

        def drain_body(t, carry):
            jr = (my - t - 1) % N_DEV
            jl = (my + t + 1) % N_DEV
            for si in range(N_SUB):
                desc_a(kg, jr, si, ksendA, krecvA, t + 1).wait_send()
                desc_a(vg, jr, si, vsendA, vrecvA, t + 1).wait_send()
                desc_b(kg, jl, si, ksendB, krecvB, t + 1).wait_send()
                desc_b(vg, jl, si, vsendB, vrecvB, t + 1).wait_send()
            return carry

        lax.fori_loop(0, N_DEV - 2, drain_body, 0)

    sem = pltpu.SemaphoreType.DMA((N_DEV - 1, N_SUB))
    out = pl.pallas_call(
        body,
        out_shape=jax.ShapeDtypeStruct((b, s, hd), jnp.float32),
        in_specs=[pl.BlockSpec(memory_space=pltpu.VMEM)] * 3,
        out_specs=pl.BlockSpec(memory_space=pltpu.VMEM),
        scratch_shapes=[
            pltpu.VMEM((b, s_glob, hd), bf16),
            pltpu.VMEM((b, s_glob, hd), bf16),
            pltpu.VMEM((b, s, hd), jnp.float32),
            pltpu.VMEM((b, s, hd), bf16),
            pltpu.VMEM((b, s, hd), bf16),
            pltpu.VMEM((b, s, hd), bf16),
            sem, sem, sem, sem,
            sem, sem, sem, sem,
        ],
        compiler_params=pltpu.CompilerParams(collective_id=0),
    )(Q2, K2, V2)
    return out.reshape(b, s, h, d)


# device time: 54425 ns/iter; 1.8602x vs baseline; 1.2501x over previous
import jax
import jax.numpy as jnp
from jax import lax
from jax.experimental import pallas as pl
from jax.experimental.pallas import tpu as pltpu

N_DEV = 8
N_SUB = 4


def kernel(Q, K, V):
    b, s, h, d = Q.shape
    hd = h * d
    half = s // 2
    sub = half // N_SUB
    scale = d ** -0.5
    Q2 = Q.reshape(b, s, hd)
    K2 = K.reshape(b, s, hd)
    V2 = V.reshape(b, s, hd)
    s_glob = N_DEV * s
    bf16 = jnp.bfloat16

    def body(q_ref, k_ref, v_ref, out_ref, kg, vg, l_scr, qs, kb, vb,
             ksendA, krecvA, vsendA, vrecvA,
             ksendB, krecvB, vsendB, vrecvB):
        my = lax.axis_index("i")
        left = (my - 1) % N_DEV
        right = (my + 1) % N_DEV

        def rows_a(j, si):
            return pl.ds(j * s + si * sub, sub)

        def rows_b(j, si):
            return pl.ds(j * s + half + si * sub, sub)

        def desc_a(ref, j, si, send_sems, recv_sems, t):
            return pltpu.make_async_remote_copy(
                src_ref=ref.at[:, rows_a(j, si), :],
                dst_ref=ref.at[:, rows_a(j, si), :],
                send_sem=send_sems.at[t, si], recv_sem=recv_sems.at[t, si],
                device_id=(right,), device_id_type=pl.DeviceIdType.MESH)

        def desc_b(ref, j, si, send_sems, recv_sems, t):
            return pltpu.make_async_remote_copy(
                src_ref=ref.at[:, rows_b(j, si), :],
                dst_ref=ref.at[:, rows_b(j, si), :],
                send_sem=send_sems.at[t, si], recv_sem=recv_sems.at[t, si],
                device_id=(left,), device_id_type=pl.DeviceIdType.MESH)

        barrier = pltpu.get_barrier_semaphore()
        for nbr in (left, right):
            pl.semaphore_signal(barrier, inc=1, device_id=(nbr,),
                                device_id_type=pl.DeviceIdType.MESH)
        pl.semaphore_wait(barrier, 2)

        kb[:, :, :] = k_ref[:, :, :].astype(bf16)
        vb[:, :, :] = v_ref[:, :, :].astype(bf16)

        own_sends = []
        for si in range(N_SUB):
            for ref_, gref, sems in ((kb, kg, (ksendA, krecvA)),
                                     (vb, vg, (vsendA, vrecvA))):
                r = pltpu.make_async_remote_copy(
                    src_ref=ref_.at[:, pl.ds(si * sub, sub), :],
                    dst_ref=gref.at[:, rows_a(my, si), :],
                    send_sem=sems[0].at[0, si], recv_sem=sems[1].at[0, si],
                    device_id=(right,), device_id_type=pl.DeviceIdType.MESH)
                r.start()
                own_sends.append(r)
            for ref_, gref, sems in ((kb, kg, (ksendB, krecvB)),
                                     (vb, vg, (vsendB, vrecvB))):
                r = pltpu.make_async_remote_copy(
                    src_ref=ref_.at[:, pl.ds(half + si * sub, sub), :],
                    dst_ref=gref.at[:, rows_b(my, si), :],
                    send_sem=sems[0].at[0, si], recv_sem=sems[1].at[0, si],
                    device_id=(left,), device_id_type=pl.DeviceIdType.MESH)
                r.start()
                own_sends.append(r)

        out_ref[:, :, :] = jnp.zeros((b, s, hd), jnp.float32)
        l_scr[:, :, :] = jnp.zeros((b, s, hd), jnp.float32)
        qs[:, :, :] = (q_ref[:, :, :] * scale).astype(bf16)

        def accum_block(bb, sl, kblk, vblk):
            q = qs[bb, :, sl]
            sb = lax.dot_general(q, kblk, (((1,), (1,)), ((), ())),
                                 preferred_element_type=jnp.float32)
            p = jnp.exp(sb)
            pv = lax.dot_general(p.astype(bf16), vblk,
                                 (((1,), (0,)), ((), ())),
                                 preferred_element_type=jnp.float32)
            out_ref[bb, :, sl] = out_ref[bb, :, sl] + pv
            l_scr[bb, :, sl] = l_scr[bb, :, sl] + jnp.sum(p, axis=1,
                                                          keepdims=True)

        def accum_block2(bb, sl, kA, kB, vA, vB):
            q = qs[bb, :, sl]
            s1 = lax.dot_general(q, kA, (((1,), (1,)), ((), ())),
                                 preferred_element_type=jnp.float32)
            s2 = lax.dot_general(q, kB, (((1,), (1,)), ((), ())),
                                 preferred_element_type=jnp.float32)
            p = jnp.exp(jnp.concatenate([s1, s2], axis=1))
            vcat = jnp.concatenate([vA, vB], axis=0)
            pv = lax.dot_general(p.astype(bf16), vcat,
                                 (((1,), (0,)), ((), ())),
                                 preferred_element_type=jnp.float32)
            out_ref[bb, :, sl] = out_ref[bb, :, sl] + pv
            l_scr[bb, :, sl] = l_scr[bb, :, sl] + jnp.sum(p, axis=1,
                                                          keepdims=True)

        for bb in range(b):
            for hh in range(h):
                sl = slice(hh * d, (hh + 1) * d)
                accum_block(bb, sl, kb[bb, :, sl], vb[bb, :, sl])

        def process_hop(t, forward):
            jr = (my - t - 1) % N_DEV
            jl = (my + t + 1) % N_DEV
            for si in range(N_SUB):
                desc_a(kg, jr, si, ksendA, krecvA, t).wait_recv()
                desc_a(vg, jr, si, vsendA, vrecvA, t).wait_recv()
                desc_b(kg, jl, si, ksendB, krecvB, t).wait_recv()
                desc_b(vg, jl, si, vsendB, vrecvB, t).wait_recv()
                if forward:
                    desc_a(kg, jr, si, ksendA, krecvA, t + 1).start()
                    desc_a(vg, jr, si, vsendA, vrecvA, t + 1).start()
                    desc_b(kg, jl, si, ksendB, krecvB, t + 1).start()
                    desc_b(vg, jl, si, vsendB, vrecvB, t + 1).start()
            for bb in range(b):
                for hh in range(h):
                    sl = slice(hh * d, (hh + 1) * d)
                    accum_block2(bb, sl,
                                 kg[bb, pl.ds(jr * s, half), sl],
                                 kg[bb, pl.ds(jl * s + half, half), sl],
                                 vg[bb, pl.ds(jr * s, half), sl],
                                 vg[bb, pl.ds(jl * s + half, half), sl])

        def hop_body(t, carry):
            process_hop(t, forward=True)
            return carry

        lax.fori_loop(0, N_DEV - 2, hop_body, 0)
        process_hop(N_DEV - 2, forward=False)

        for bb in range(b):
            out_ref[bb, :, :] = out_ref[bb, :, :] / l_scr[bb, :, :]

        for r in own_sends:
            r.wait_send()

        def drain_body(t, carry):
            jr = (my - t - 1) % N_DEV
            jl = (my + t + 1) % N_DEV
            for si in range(N_SUB):
                desc_a(kg, jr, si, ksendA, krecvA, t + 1).wait_send()
                desc_a(vg, jr, si, vsendA, vrecvA, t + 1).wait_send()
                desc_b(kg, jl, si, ksendB, krecvB, t + 1).wait_send()
                desc_b(vg, jl, si, vsendB, vrecvB, t + 1).wait_send()
            return carry

        lax.fori_loop(0, N_DEV - 2, drain_body, 0)

    sem = pltpu.SemaphoreType.DMA((N_DEV - 1, N_SUB))
    out = pl.pallas_call(
        body,
        out_shape=jax.ShapeDtypeStruct((b, s, hd), jnp.float32),
        in_specs=[pl.BlockSpec(memory_space=pltpu.VMEM)] * 3,
        out_specs=pl.BlockSpec(memory_space=pltpu.VMEM),
        scratch_shapes=[
            pltpu.VMEM((b, s_glob, hd), bf16),
            pltpu.VMEM((b, s_glob, hd), bf16),
            pltpu.VMEM((b, s, hd), jnp.float32),
            pltpu.VMEM((b, s, hd), bf16),
            pltpu.VMEM((b, s, hd), bf16),
            pltpu.VMEM((b, s, hd), bf16),
            sem, sem, sem, sem,
            sem, sem, sem, sem,
        ],
        compiler_params=pltpu.CompilerParams(collective_id=0),
    )(Q2, K2, V2)
    return out.reshape(b, s, h, d)


# device time: 53382 ns/iter; 1.8966x vs baseline; 1.0195x over previous
import jax
import jax.numpy as jnp
from jax import lax
from jax.experimental import pallas as pl
from jax.experimental.pallas import tpu as pltpu

N_DEV = 8
N_SUB = 2


def kernel(Q, K, V):
    b, s, h, d = Q.shape
    hd = h * d
    half = s // 2
    sub = half // N_SUB
    scale = d ** -0.5
    Q2 = Q.reshape(b, s, hd)
    K2 = K.reshape(b, s, hd)
    V2 = V.reshape(b, s, hd)
    s_glob = N_DEV * s
    bf16 = jnp.bfloat16

    def body(q_ref, k_ref, v_ref, out_ref, kg, vg, l_scr, qs, kb, vb,
             ksendA, krecvA, vsendA, vrecvA,
             ksendB, krecvB, vsendB, vrecvB):
        my = lax.axis_index("i")
        left = (my - 1) % N_DEV
        right = (my + 1) % N_DEV

        def rows_a(j, si):
            return pl.ds(j * s + si * sub, sub)

        def rows_b(j, si):
            return pl.ds(j * s + half + si * sub, sub)

        def desc_a(ref, j, si, send_sems, recv_sems, t):
            return pltpu.make_async_remote_copy(
                src_ref=ref.at[:, rows_a(j, si), :],
                dst_ref=ref.at[:, rows_a(j, si), :],
                send_sem=send_sems.at[t, si], recv_sem=recv_sems.at[t, si],
                device_id=(right,), device_id_type=pl.DeviceIdType.MESH)

        def desc_b(ref, j, si, send_sems, recv_sems, t):
            return pltpu.make_async_remote_copy(
                src_ref=ref.at[:, rows_b(j, si), :],
                dst_ref=ref.at[:, rows_b(j, si), :],
                send_sem=send_sems.at[t, si], recv_sem=recv_sems.at[t, si],
                device_id=(left,), device_id_type=pl.DeviceIdType.MESH)

        barrier = pltpu.get_barrier_semaphore()
        for nbr in (left, right):
            pl.semaphore_signal(barrier, inc=1, device_id=(nbr,),
                                device_id_type=pl.DeviceIdType.MESH)
        pl.semaphore_wait(barrier, 2)

        kb[:, :, :] = k_ref[:, :, :].astype(bf16)
        vb[:, :, :] = v_ref[:, :, :].astype(bf16)

        own_sends = []
        for si in range(N_SUB):
            for ref_, gref, sems in ((kb, kg, (ksendA, krecvA)),
                                     (vb, vg, (vsendA, vrecvA))):
                r = pltpu.make_async_remote_copy(
                    src_ref=ref_.at[:, pl.ds(si * sub, sub), :],
                    dst_ref=gref.at[:, rows_a(my, si), :],
                    send_sem=sems[0].at[0, si], recv_sem=sems[1].at[0, si],
                    device_id=(right,), device_id_type=pl.DeviceIdType.MESH)
                r.start()
                own_sends.append(r)
            for ref_, gref, sems in ((kb, kg, (ksendB, krecvB)),
                                     (vb, vg, (vsendB, vrecvB))):
                r = pltpu.make_async_remote_copy(
                    src_ref=ref_.at[:, pl.ds(half + si * sub, sub), :],
                    dst_ref=gref.at[:, rows_b(my, si), :],
                    send_sem=sems[0].at[0, si], recv_sem=sems[1].at[0, si],
                    device_id=(left,), device_id_type=pl.DeviceIdType.MESH)
                r.start()
                own_sends.append(r)

        out_ref[:, :, :] = jnp.zeros((b, s, hd), jnp.float32)
        l_scr[:, :, :] = jnp.zeros((b, s, hd), jnp.float32)
        qs[:, :, :] = (q_ref[:, :, :] * scale).astype(bf16)

        def accum_block(bb, sl, kblk, vblk):
            q = qs[bb, :, sl]
            sb = lax.dot_general(q, kblk, (((1,), (1,)), ((), ())),
                                 preferred_element_type=jnp.float32)
            p = jnp.exp(sb)
            pv = lax.dot_general(p.astype(bf16), vblk,
                                 (((1,), (0,)), ((), ())),
                                 preferred_element_type=jnp.float32)
            out_ref[bb, :, sl] = out_ref[bb, :, sl] + pv
            l_scr[bb, :, sl] = l_scr[bb, :, sl] + jnp.sum(p, axis=1,
                                                          keepdims=True)

        def accum_block2(bb, sl, kA, kB, vA, vB):
            q = qs[bb, :, sl]
            s1 = lax.dot_general(q, kA, (((1,), (1,)), ((), ())),
                                 preferred_element_type=jnp.float32)
            s2 = lax.dot_general(q, kB, (((1,), (1,)), ((), ())),
                                 preferred_element_type=jnp.float32)
            p = jnp.exp(jnp.concatenate([s1, s2], axis=1))
            vcat = jnp.concatenate([vA, vB], axis=0)
            pv = lax.dot_general(p.astype(bf16), vcat,
                                 (((1,), (0,)), ((), ())),
                                 preferred_element_type=jnp.float32)
            out_ref[bb, :, sl] = out_ref[bb, :, sl] + pv
            l_scr[bb, :, sl] = l_scr[bb, :, sl] + jnp.sum(p, axis=1,
                                                          keepdims=True)

        for bb in range(b):
            for hh in range(h):
                sl = slice(hh * d, (hh + 1) * d)
                accum_block(bb, sl, kb[bb, :, sl], vb[bb, :, sl])

        def process_hop(t, forward):
            jr = (my - t - 1) % N_DEV
            jl = (my + t + 1) % N_DEV
            for si in range(N_SUB):
                desc_a(kg, jr, si, ksendA, krecvA, t).wait_recv()
                desc_a(vg, jr, si, vsendA, vrecvA, t).wait_recv()
                desc_b(kg, jl, si, ksendB, krecvB, t).wait_recv()
                desc_b(vg, jl, si, vsendB, vrecvB, t).wait_recv()
                if forward:
                    desc_a(kg, jr, si, ksendA, krecvA, t + 1).start()
                    desc_a(vg, jr, si, vsendA, vrecvA, t + 1).start()
                    desc_b(kg, jl, si, ksendB, krecvB, t + 1).start()
                    desc_b(vg, jl, si, vsendB, vrecvB, t + 1).start()
            for bb in range(b):
                for hh in range(h):
                    sl = slice(hh * d, (hh + 1) * d)
                    accum_block2(bb, sl,
                                 kg[bb, pl.ds(jr * s, half), sl],
                                 kg[bb, pl.ds(jl * s + half, half), sl],
                                 vg[bb, pl.ds(jr * s, half), sl],
                                 vg[bb, pl.ds(jl * s + half, half), sl])

        def hop_body(t, carry):
            process_hop(t, forward=True)
            return carry

        lax.fori_loop(0, N_DEV - 2, hop_body, 0)
        process_hop(N_DEV - 2, forward=False)

        for bb in range(b):
            out_ref[bb, :, :] = out_ref[bb, :, :] / l_scr[bb, :, :]

        for r in own_sends:
            r.wait_send()

        def drain_body(t, carry):
            jr = (my - t - 1) % N_DEV
            jl = (my + t + 1) % N_DEV
            for si in range(N_SUB):
                desc_a(kg, jr, si, ksendA, krecvA, t + 1).wait_send()
                desc_a(vg, jr, si, vsendA, vrecvA, t + 1).wait_send()
                desc_b(kg, jl, si, ksendB, krecvB, t + 1).wait_send()
                desc_b(vg, jl, si, vsendB, vrecvB, t + 1).wait_send()
            return carry

        lax.fori_loop(0, N_DEV - 2, drain_body, 0)

    sem = pltpu.SemaphoreType.DMA((N_DEV - 1, N_SUB))
    out = pl.pallas_call(
        body,
        out_shape=jax.ShapeDtypeStruct((b, s, hd), jnp.float32),
        in_specs=[pl.BlockSpec(memory_space=pltpu.VMEM)] * 3,
        out_specs=pl.BlockSpec(memory_space=pltpu.VMEM),
        scratch_shapes=[
            pltpu.VMEM((b, s_glob, hd), bf16),
            pltpu.VMEM((b, s_glob, hd), bf16),
            pltpu.VMEM((b, s, hd), jnp.float32),
            pltpu.VMEM((b, s, hd), bf16),
            pltpu.VMEM((b, s, hd), bf16),
            pltpu.VMEM((b, s, hd), bf16),
            sem, sem, sem, sem,
            sem, sem, sem, sem,
        ],
        compiler_params=pltpu.CompilerParams(collective_id=0),
    )(Q2, K2, V2)
    return out.reshape(b, s, h, d)


# device time: 38079 ns/iter; 2.6588x vs baseline; 1.4019x over previous
import jax
import jax.numpy as jnp
from jax import lax
from jax.experimental import pallas as pl
from jax.experimental.pallas import tpu as pltpu

N_DEV = 8
N_SUB = 2


def kernel(Q, K, V):
    b, s, h, d = Q.shape
    hd = h * d
    half = s // 2
    sub = half // N_SUB
    scale = d ** -0.5
    Q2 = Q.reshape(b, s, hd)
    K2 = K.reshape(b, s, hd)
    V2 = V.reshape(b, s, hd)
    s_glob = N_DEV * s
    bf16 = jnp.bfloat16
    wire_t = jnp.float8_e4m3fn

    def body(q_ref, k_ref, v_ref, out_ref, kg, vg, l_scr, qs, kb, vb,
             ksendA, krecvA, vsendA, vrecvA,
             ksendB, krecvB, vsendB, vrecvB):
        my = lax.axis_index("i")
        left = (my - 1) % N_DEV
        right = (my + 1) % N_DEV

        def rows_a(j, si):
            return pl.ds(j * s + si * sub, sub)

        def rows_b(j, si):
            return pl.ds(j * s + half + si * sub, sub)

        def desc_a(ref, j, si, send_sems, recv_sems, t):
            return pltpu.make_async_remote_copy(
                src_ref=ref.at[:, rows_a(j, si), :],
                dst_ref=ref.at[:, rows_a(j, si), :],
                send_sem=send_sems.at[t, si], recv_sem=recv_sems.at[t, si],
                device_id=(right,), device_id_type=pl.DeviceIdType.MESH)

        def desc_b(ref, j, si, send_sems, recv_sems, t):
            return pltpu.make_async_remote_copy(
                src_ref=ref.at[:, rows_b(j, si), :],
                dst_ref=ref.at[:, rows_b(j, si), :],
                send_sem=send_sems.at[t, si], recv_sem=recv_sems.at[t, si],
                device_id=(left,), device_id_type=pl.DeviceIdType.MESH)

        barrier = pltpu.get_barrier_semaphore()
        for nbr in (left, right):
            pl.semaphore_signal(barrier, inc=1, device_id=(nbr,),
                                device_id_type=pl.DeviceIdType.MESH)
        pl.semaphore_wait(barrier, 2)

        kb[:, :, :] = k_ref[:, :, :].astype(wire_t)
        vb[:, :, :] = v_ref[:, :, :].astype(wire_t)

        own_sends = []
        for si in range(N_SUB):
            for ref_, gref, sems in ((kb, kg, (ksendA, krecvA)),
                                     (vb, vg, (vsendA, vrecvA))):
                r = pltpu.make_async_remote_copy(
                    src_ref=ref_.at[:, pl.ds(si * sub, sub), :],
                    dst_ref=gref.at[:, rows_a(my, si), :],
                    send_sem=sems[0].at[0, si], recv_sem=sems[1].at[0, si],
                    device_id=(right,), device_id_type=pl.DeviceIdType.MESH)
                r.start()
                own_sends.append(r)
            for ref_, gref, sems in ((kb, kg, (ksendB, krecvB)),
                                     (vb, vg, (vsendB, vrecvB))):
                r = pltpu.make_async_remote_copy(
                    src_ref=ref_.at[:, pl.ds(half + si * sub, sub), :],
                    dst_ref=gref.at[:, rows_b(my, si), :],
                    send_sem=sems[0].at[0, si], recv_sem=sems[1].at[0, si],
                    device_id=(left,), device_id_type=pl.DeviceIdType.MESH)
                r.start()
                own_sends.append(r)

        out_ref[:, :, :] = jnp.zeros((b, s, hd), jnp.float32)
        l_scr[:, :, :] = jnp.zeros((b, s, hd), jnp.float32)
        qs[:, :, :] = (q_ref[:, :, :] * scale).astype(bf16)

        def accum_block(bb, sl, kblk, vblk):
            q = qs[bb, :, sl]
            kblk = kblk.astype(bf16)
            vblk = vblk.astype(bf16)
            sb = lax.dot_general(q, kblk, (((1,), (1,)), ((), ())),
                                 preferred_element_type=jnp.float32)
            p = jnp.exp(sb)
            pv = lax.dot_general(p.astype(bf16), vblk,
                                 (((1,), (0,)), ((), ())),
                                 preferred_element_type=jnp.float32)
            out_ref[bb, :, sl] = out_ref[bb, :, sl] + pv
            l_scr[bb, :, sl] = l_scr[bb, :, sl] + jnp.sum(p, axis=1,
                                                          keepdims=True)

        def accum_block2(bb, sl, kA, kB, vA, vB):
            q = qs[bb, :, sl]
            kA = kA.astype(bf16)
            kB = kB.astype(bf16)
            s1 = lax.dot_general(q, kA, (((1,), (1,)), ((), ())),
                                 preferred_element_type=jnp.float32)
            s2 = lax.dot_general(q, kB, (((1,), (1,)), ((), ())),
                                 preferred_element_type=jnp.float32)
            p = jnp.exp(jnp.concatenate([s1, s2], axis=1))
            vcat = jnp.concatenate([vA, vB], axis=0).astype(bf16)
            pv = lax.dot_general(p.astype(bf16), vcat,
                                 (((1,), (0,)), ((), ())),
                                 preferred_element_type=jnp.float32)
            out_ref[bb, :, sl] = out_ref[bb, :, sl] + pv
            l_scr[bb, :, sl] = l_scr[bb, :, sl] + jnp.sum(p, axis=1,
                                                          keepdims=True)

        for bb in range(b):
            for hh in range(h):
                sl = slice(hh * d, (hh + 1) * d)
                accum_block(bb, sl, kb[bb, :, sl], vb[bb, :, sl])

        def process_hop(t, forward):
            jr = (my - t - 1) % N_DEV
            jl = (my + t + 1) % N_DEV
            for si in range(N_SUB):
                desc_a(kg, jr, si, ksendA, krecvA, t).wait_recv()
                desc_a(vg, jr, si, vsendA, vrecvA, t).wait_recv()
                desc_b(kg, jl, si, ksendB, krecvB, t).wait_recv()
                desc_b(vg, jl, si, vsendB, vrecvB, t).wait_recv()
                if forward:
                    desc_a(kg, jr, si, ksendA, krecvA, t + 1).start()
                    desc_a(vg, jr, si, vsendA, vrecvA, t + 1).start()
                    desc_b(kg, jl, si, ksendB, krecvB, t + 1).start()
                    desc_b(vg, jl, si, vsendB, vrecvB, t + 1).start()
            for bb in range(b):
                for hh in range(h):
                    sl = slice(hh * d, (hh + 1) * d)
                    accum_block2(bb, sl,
                                 kg[bb, pl.ds(jr * s, half), sl],
                                 kg[bb, pl.ds(jl * s + half, half), sl],
                                 vg[bb, pl.ds(jr * s, half), sl],
                                 vg[bb, pl.ds(jl * s + half, half), sl])

        def hop_body(t, carry):
            process_hop(t, forward=True)
            return carry

        lax.fori_loop(0, N_DEV - 2, hop_body, 0)
        process_hop(N_DEV - 2, forward=False)

        for bb in range(b):
            out_ref[bb, :, :] = out_ref[bb, :, :] / l_scr[bb, :, :]

        for r in own_sends:
            r.wait_send()

        def drain_body(t, carry):
            jr = (my - t - 1) % N_DEV
            jl = (my + t + 1) % N_DEV
            for si in range(N_SUB):
                desc_a(kg, jr, si, ksendA, krecvA, t + 1).wait_send()
                desc_a(vg, jr, si, vsendA, vrecvA, t + 1).wait_send()
                desc_b(kg, jl, si, ksendB, krecvB, t + 1).wait_send()
                desc_b(vg, jl, si, vsendB, vrecvB, t + 1).wait_send()
            return carry

        lax.fori_loop(0, N_DEV - 2, drain_body, 0)

    sem = pltpu.SemaphoreType.DMA((N_DEV - 1, N_SUB))
    out = pl.pallas_call(
        body,
        out_shape=jax.ShapeDtypeStruct((b, s, hd), jnp.float32),
        in_specs=[pl.BlockSpec(memory_space=pltpu.VMEM)] * 3,
        out_specs=pl.BlockSpec(memory_space=pltpu.VMEM),
        scratch_shapes=[
            pltpu.VMEM((b, s_glob, hd), wire_t),
            pltpu.VMEM((b, s_glob, hd), wire_t),
            pltpu.VMEM((b, s, hd), jnp.float32),
            pltpu.VMEM((b, s, hd), bf16),
            pltpu.VMEM((b, s, hd), wire_t),
            pltpu.VMEM((b, s, hd), wire_t),
            sem, sem, sem, sem,
            sem, sem, sem, sem,
        ],
        compiler_params=pltpu.CompilerParams(collective_id=0),
    )(Q2, K2, V2)
    return out.reshape(b, s, h, d)
